# baseline (speedup 1.0000x reference)
import functools

import jax
import jax.numpy as jnp
from jax import lax
from jax.experimental import pallas as pl
from jax.experimental.pallas import tpu as pltpu
from jax.experimental.pallas import tpu_sc as plsc

BATCH = 16384
D = 32
NUSERS = 1000000
NC = 2
NS = 16
NW = NC * NS
BPW = BATCH // NW
L = 16
NG = BPW // L
NROW = D * BPW // 128

_mesh = plsc.VectorSubcoreMesh(core_axis_name="c", subcore_axis_name="s")


@functools.partial(
    pl.kernel,
    mesh=_mesh,
    compiler_params=pltpu.CompilerParams(
        needs_layout_passes=False, use_tc_tiling_on_sc=False),
    out_type=jax.ShapeDtypeStruct((BATCH,), jnp.float32),
    scratch_types=[
        pltpu.VMEM((BPW,), jnp.int32),
        pltpu.VMEM((BPW,), jnp.int32),
        pltpu.VMEM((NROW, 128), jnp.int32),
        pltpu.VMEM((NROW, 128), jnp.int32),
        pltpu.VMEM((NROW, 128), jnp.float32),
        pltpu.VMEM((NROW, 128), jnp.float32),
        pltpu.VMEM((BPW,), jnp.float32),
        pltpu.SemaphoreType.DMA,
    ],
)
def _dot_sc(uid_hbm, lid_hbm, utab_hbm, ltab_hbm, out_hbm,
            uidx_v, lidx_v, uaddr_v, laddr_v, uval_v, lval_v, out_v, sem):
    wid = lax.axis_index("s") * NC + lax.axis_index("c")
    base = wid * BPW

    pltpu.sync_copy(uid_hbm.at[pl.ds(base, BPW)], uidx_v)
    pltpu.sync_copy(lid_hbm.at[pl.ds(base, BPW)], lidx_v)

    def body_addr(g, carry):
        uids = uidx_v[pl.ds(g * L, L)]
        lids = lidx_v[pl.ds(g * L, L)]
        gdiv = g >> 3
        col = (g & 7) * L
        for d in range(D):
            uaddr_v[4 * d + gdiv, pl.ds(col, L)] = uids + (d * NUSERS)
            laddr_v[4 * d + gdiv, pl.ds(col, L)] = lids + (d * NUSERS)
        return carry

    lax.fori_loop(0, NG, body_addr, 0)

    def body_fire(j, carry):
        pltpu.async_copy(utab_hbm.at[uaddr_v.at[j]], uval_v.at[j], sem)
        pltpu.async_copy(ltab_hbm.at[laddr_v.at[j]], lval_v.at[j], sem)
        return carry

    lax.fori_loop(0, NROW, body_fire, 0)

    def body_drain(j, carry):
        pltpu.make_async_copy(utab_hbm.at[uaddr_v.at[j]], uval_v.at[j],
                              sem).wait()
        pltpu.make_async_copy(ltab_hbm.at[laddr_v.at[j]], lval_v.at[j],
                              sem).wait()
        return carry

    lax.fori_loop(0, NROW, body_drain, 0)

    def body_dot(g, carry):
        gdiv = g >> 3
        col = (g & 7) * L
        acc = jnp.zeros((L,), jnp.float32)
        for d in range(D):
            u = uval_v[4 * d + gdiv, pl.ds(col, L)]
            v = lval_v[4 * d + gdiv, pl.ds(col, L)]
            acc = acc + u * v
        out_v[pl.ds(g * L, L)] = acc
        return carry

    lax.fori_loop(0, NG, body_dot, 0)

    pltpu.sync_copy(out_v, out_hbm.at[pl.ds(base, BPW)])


def kernel(user_id, lawyer_id, user_table, lawyer_table):
    utab_flat = user_table.T.reshape(-1)
    ltab_flat = lawyer_table.T.reshape(-1)
    out = _dot_sc(user_id.astype(jnp.int32), lawyer_id.astype(jnp.int32),
                  utab_flat, ltab_flat)
    return out.reshape(BATCH, 1)


# speedup vs baseline: 5.7012x; 5.7012x over previous
import functools

import jax
import jax.numpy as jnp
from jax import lax
from jax.experimental import pallas as pl
from jax.experimental.pallas import tpu as pltpu
from jax.experimental.pallas import tpu_sc as plsc

BATCH = 16384
D = 32
GROUPS = 250000
NC = 2
NS = 16
NW = NC * NS
BPW = BATCH // NW
HALF = BPW // 2
CHUNK = 128
L = 16

_mesh = plsc.VectorSubcoreMesh(core_axis_name="c", subcore_axis_name="s")


@functools.partial(
    pl.kernel,
    mesh=_mesh,
    compiler_params=pltpu.CompilerParams(
        needs_layout_passes=False, use_tc_tiling_on_sc=False),
    out_type=jax.ShapeDtypeStruct((BATCH,), jnp.float32),
    scratch_types=[
        pltpu.VMEM((BPW,), jnp.int32),
        pltpu.VMEM((BPW,), jnp.int32),
        pltpu.VMEM((BPW,), jnp.int32),
        pltpu.VMEM((BPW,), jnp.int32),
        pltpu.VMEM((HALF, 128), jnp.float32),
        pltpu.VMEM((HALF, 128), jnp.float32),
        pltpu.VMEM((BPW,), jnp.float32),
        pltpu.SemaphoreType.DMA,
    ],
)
def _dot_sc(uid_hbm, lid_hbm, utab_hbm, ltab_hbm, out_hbm,
            uidx_v, lidx_v, ugrp_v, lgrp_v, urows_v, lrows_v, out_v, sem):
    wid = lax.axis_index("s") * NC + lax.axis_index("c")
    base = wid * BPW

    pltpu.sync_copy(uid_hbm.at[pl.ds(base, BPW)], uidx_v)
    pltpu.sync_copy(lid_hbm.at[pl.ds(base, BPW)], lidx_v)

    def body_grp(g, carry):
        sl = pl.ds(g * L, L)
        ugrp_v[sl] = uidx_v[sl] >> 2
        lgrp_v[sl] = lidx_v[sl] >> 2
        return carry

    lax.fori_loop(0, BPW // L, body_grp, 0)

    iota = lax.iota(jnp.int32, L)

    for half in range(2):
        hbase = half * HALF
        copies = []
        for j in range(HALF // CHUNK):
            isl = pl.ds(hbase + j * CHUNK, CHUNK)
            osl = pl.ds(j * CHUNK, CHUNK)
            copies.append(pltpu.async_copy(
                utab_hbm.at[ugrp_v.at[isl]], urows_v.at[osl], sem))
            copies.append(pltpu.async_copy(
                ltab_hbm.at[lgrp_v.at[isl]], lrows_v.at[osl], sem))
        for cp in copies:
            cp.wait()

        def body(g, carry):
            acc = jnp.zeros((L,), jnp.float32)
            usub = (uidx_v[pl.ds(hbase + g * L, L)] & 3) * D
            lsub = (lidx_v[pl.ds(hbase + g * L, L)] & 3) * D
            for j in range(L):
                r = g * L + j
                ucol = usub[j]
                lcol = lsub[j]
                u0 = urows_v[r, pl.ds(ucol, L)]
                u1 = urows_v[r, pl.ds(ucol + L, L)]
                l0 = lrows_v[r, pl.ds(lcol, L)]
                l1 = lrows_v[r, pl.ds(lcol + L, L)]
                h = u0 * l0 + u1 * l1
                s = jnp.sum(h)
                acc = acc + jnp.where(iota == j, s, jnp.float32(0.0))
            out_v[pl.ds(hbase + g * L, L)] = acc
            return carry

        lax.fori_loop(0, HALF // L, body, 0)

    pltpu.sync_copy(out_v, out_hbm.at[pl.ds(base, BPW)])


def kernel(user_id, lawyer_id, user_table, lawyer_table):
    out = _dot_sc(user_id.astype(jnp.int32), lawyer_id.astype(jnp.int32),
                  user_table.reshape(GROUPS, 128),
                  lawyer_table.reshape(GROUPS, 128))
    return out.reshape(BATCH, 1)


# speedup vs baseline: 5.7336x; 1.0057x over previous
import functools

import jax
import jax.numpy as jnp
from jax import lax
from jax.experimental import pallas as pl
from jax.experimental.pallas import tpu as pltpu
from jax.experimental.pallas import tpu_sc as plsc

BATCH = 16384
D = 32
NC = 2
NS = 16
NW = NC * NS
BPW = BATCH // NW
CHUNK = 128
NCHUNK = BPW // CHUNK
L = 16

_mesh = plsc.VectorSubcoreMesh(core_axis_name="c", subcore_axis_name="s")


@functools.partial(
    pl.kernel,
    mesh=_mesh,
    compiler_params=pltpu.CompilerParams(
        needs_layout_passes=False, use_tc_tiling_on_sc=False),
    out_type=jax.ShapeDtypeStruct((BATCH,), jnp.float32),
    scratch_types=[
        pltpu.VMEM((BPW,), jnp.int32),
        pltpu.VMEM((BPW,), jnp.int32),
        pltpu.VMEM((BPW, D), jnp.float32),
        pltpu.VMEM((BPW, D), jnp.float32),
        pltpu.VMEM((BPW,), jnp.float32),
        pltpu.SemaphoreType.DMA,
    ],
)
def _dot_sc(uid_hbm, lid_hbm, utab_hbm, ltab_hbm, out_hbm,
            uidx_v, lidx_v, urows_v, lrows_v, out_v, sem):
    wid = lax.axis_index("s") * NC + lax.axis_index("c")
    base = wid * BPW

    pltpu.sync_copy(uid_hbm.at[pl.ds(base, BPW)], uidx_v)
    pltpu.sync_copy(lid_hbm.at[pl.ds(base, BPW)], lidx_v)

    copies = []
    for j in range(NCHUNK):
        sl = pl.ds(j * CHUNK, CHUNK)
        copies.append(pltpu.async_copy(
            utab_hbm.at[uidx_v.at[sl]], urows_v.at[sl], sem))
        copies.append(pltpu.async_copy(
            ltab_hbm.at[lidx_v.at[sl]], lrows_v.at[sl], sem))
    for cp in copies:
        cp.wait()

    iota = lax.iota(jnp.int32, L)

    def body(g, carry):
        acc = jnp.zeros((L,), jnp.float32)
        for j in range(L):
            r = g * L + j
            u0 = urows_v[r, pl.ds(0, L)]
            u1 = urows_v[r, pl.ds(L, L)]
            l0 = lrows_v[r, pl.ds(0, L)]
            l1 = lrows_v[r, pl.ds(L, L)]
            h = u0 * l0 + u1 * l1
            s = jnp.sum(h)
            acc = acc + jnp.where(iota == j, s, jnp.float32(0.0))
        out_v[pl.ds(g * L, L)] = acc
        return carry

    lax.fori_loop(0, BPW // L, body, 0)

    pltpu.sync_copy(out_v, out_hbm.at[pl.ds(base, BPW)])


def kernel(user_id, lawyer_id, user_table, lawyer_table):
    out = _dot_sc(user_id.astype(jnp.int32), lawyer_id.astype(jnp.int32),
                  user_table, lawyer_table)
    return out.reshape(BATCH, 1)
